# baseline (speedup 1.0000x reference)
import jax
import jax.numpy as jnp
from jax import lax
from jax.experimental import pallas as pl
from jax.experimental.pallas import tpu as pltpu
from jax.experimental.pallas import tpu_sc as plsc

N_PAD = 50176
ROWS = N_PAD // 128
E_TOTAL = 1600000
CHUNKS = E_TOTAL // 128
BLK = 1024
CPB = BLK // 128
NFULL = E_TOTAL // BLK
TAIL_BID = NFULL
TAIL_VALID_FROM = NFULL * BLK - (E_TOTAL - BLK)
NW = 32
NVALID_ALL = NFULL + 1
KMAX = (NVALID_ALL + NW - 1) // NW
PER_TILE_SLICE = N_PAD // 16
DUMP_NODE = N_PAD - 1


def _node_xy_body(v_ref, th_ref, x_ref, y_ref):
    m = jnp.abs(v_ref[...])
    th = th_ref[...]
    x_ref[...] = m * jnp.cos(th)
    y_ref[...] = m * jnp.sin(th)


def _node_xy(v2, th2):
    return pl.pallas_call(
        _node_xy_body,
        out_shape=(
            jax.ShapeDtypeStruct((ROWS, 128), jnp.float32),
            jax.ShapeDtypeStruct((ROWS, 128), jnp.float32),
        ),
    )(v2, th2)


def _edge_body(x_hbm, y_hbm, ef_hbm, af_hbm,
               pact0_hbm, pact1_hbm, prea0_hbm, prea1_hbm,
               xv, yv, eiv0, eiv1, afv0, afv1, fiv0, fiv1,
               actv0, actv1, reav0, reav1,
               in_sem0, in_sem1, sc_sem0, sc_sem1,
               acc_act, acc_rea):
    eiv = [eiv0, eiv1]
    afv = [afv0, afv1]
    fiv = [fiv0, fiv1]
    actv = [actv0, actv1]
    reav = [reav0, reav1]
    in_sem = [in_sem0, in_sem1]
    sc_sem = [sc_sem0, sc_sem1]

    cid = lax.axis_index("c")
    sid = lax.axis_index("s")
    wid = cid * 16 + sid

    def _z(i, c):
        xv[pl.ds(i * 16, 16)] = jnp.zeros((16,), jnp.float32)
        return c
    lax.fori_loop(0, PER_TILE_SLICE // 16, _z, 0)
    pltpu.sync_copy(xv.at[pl.ds(0, PER_TILE_SLICE)],
                    acc_act.at[pl.ds(sid * PER_TILE_SLICE, PER_TILE_SLICE)])
    pltpu.sync_copy(xv.at[pl.ds(0, PER_TILE_SLICE)],
                    acc_rea.at[pl.ds(sid * PER_TILE_SLICE, PER_TILE_SLICE)])

    lanes = lax.iota(jnp.int32, 16)

    def _base(k):
        bid = wid + NW * k
        return jnp.minimum(bid * BLK, E_TOTAL - BLK)

    def _fire_inputs(k):
        b = _base(k)
        s = in_sem[k % 2]
        pltpu.async_copy(ef_hbm.at[pl.ds(b * 2, BLK * 2)], eiv[k % 2], s)
        pltpu.async_copy(af_hbm.at[pl.ds(b * 4, BLK * 4)], afv[k % 2], s)

    def _wait_inputs(k):
        s = in_sem[k % 2]
        pltpu.make_async_copy(
            ef_hbm.at[pl.ds(0, BLK * 2)], eiv[k % 2], s).wait()
        pltpu.make_async_copy(
            af_hbm.at[pl.ds(0, BLK * 4)], afv[k % 2], s).wait()

    def _fire_scatter(k):
        s = sc_sem[k % 2]
        pltpu.async_copy(actv[k % 2], acc_act.at[fiv[k % 2]], s, add=True)
        pltpu.async_copy(reav[k % 2], acc_rea.at[fiv[k % 2]], s, add=True)

    def _wait_scatter(k):
        s = sc_sem[k % 2]
        pltpu.make_async_copy(actv[k % 2], acc_act.at[fiv[k % 2]], s).wait()
        pltpu.make_async_copy(reav[k % 2], acc_rea.at[fiv[k % 2]], s).wait()

    def _compute(k, tail):
        ev, av = eiv[k % 2], afv[k % 2]
        fv, ov, rv = fiv[k % 2], actv[k % 2], reav[k % 2]

        def _chunk(j8, c):
            for i2 in range(8):
                le = j8 * 256 + i2 * 16
                la = j8 * 512 + i2 * 16
                lo = j8 * 128 + i2 * 16
                fi = ev[pl.ds(le, 16)]
                ti = ev[pl.ds(le + 128, 16)]
                if tail:
                    ok = (lo + lanes) >= TAIL_VALID_FROM
                    fi = jnp.where(ok, fi, DUMP_NODE)
                fv[pl.ds(lo, 16)] = fi
                xf = plsc.load_gather(xv, [fi])
                yf = plsc.load_gather(yv, [fi])
                xt = plsc.load_gather(xv, [ti])
                yt = plsc.load_gather(yv, [ti])
                a0 = av[pl.ds(la, 16)]
                a1 = av[pl.ds(la + 128, 16)]
                p = xf * xt + yf * yt
                q = yf * xt - xf * yt
                act = a0 * p + a1 * q
                rea = a0 * q - a1 * p
                if tail:
                    act = jnp.where(ok, act, 0.0)
                    rea = jnp.where(ok, rea, 0.0)
                ov[pl.ds(lo, 16)] = act
                rv[pl.ds(lo, 16)] = rea
            return c
        lax.fori_loop(0, CPB, _chunk, 0)

    _fire_inputs(0)
    pltpu.sync_copy(x_hbm, xv)
    pltpu.sync_copy(y_hbm, yv)
    plsc.subcore_barrier()

    for k in range(KMAX):
        last = k == KMAX - 1
        bid = wid + NW * k
        valid = bid < NVALID_ALL

        if not last:
            _wait_inputs(k)
            if k >= 2:
                _wait_scatter(k - 2)
            if k + 1 < KMAX - 1:
                _fire_inputs(k + 1)
            else:
                @pl.when(wid + NW * (KMAX - 1) < NVALID_ALL)
                def _():
                    _fire_inputs(KMAX - 1)
            _compute(k, tail=False)
            _fire_scatter(k)
        else:
            @pl.when(valid)
            def _():
                _wait_inputs(k)
            _wait_scatter(k - 2)

            @pl.when(bid < NFULL)
            def _():
                _compute(k, tail=False)

            @pl.when(bid == TAIL_BID)
            def _():
                _compute(k, tail=True)

            @pl.when(valid)
            def _():
                _fire_scatter(k)

    _wait_scatter(KMAX - 2)

    @pl.when(wid + NW * (KMAX - 1) < NVALID_ALL)
    def _():
        _wait_scatter(KMAX - 1)

    plsc.subcore_barrier()

    sl = pl.ds(sid * PER_TILE_SLICE, PER_TILE_SLICE)
    tsl = pl.ds(0, PER_TILE_SLICE)
    pltpu.sync_copy(acc_act.at[sl], xv.at[tsl])
    pltpu.sync_copy(acc_rea.at[sl], yv.at[tsl])

    @pl.when(cid == 0)
    def _():
        pltpu.sync_copy(xv.at[tsl], pact0_hbm.at[sl])
        pltpu.sync_copy(yv.at[tsl], prea0_hbm.at[sl])

    @pl.when(cid == 1)
    def _():
        pltpu.sync_copy(xv.at[tsl], pact1_hbm.at[sl])
        pltpu.sync_copy(yv.at[tsl], prea1_hbm.at[sl])


def _edge_accumulate(x1, y1, eflat, aflat):
    mesh = plsc.VectorSubcoreMesh(core_axis_name="c", subcore_axis_name="s")
    f = pl.kernel(
        _edge_body,
        out_type=tuple(
            jax.ShapeDtypeStruct((N_PAD,), jnp.float32) for _ in range(4)),
        mesh=mesh,
        compiler_params=pltpu.CompilerParams(needs_layout_passes=False),
        scratch_types=[
            pltpu.VMEM((N_PAD,), jnp.float32),
            pltpu.VMEM((N_PAD,), jnp.float32),
            pltpu.VMEM((BLK * 2,), jnp.int32),
            pltpu.VMEM((BLK * 2,), jnp.int32),
            pltpu.VMEM((BLK * 4,), jnp.float32),
            pltpu.VMEM((BLK * 4,), jnp.float32),
            pltpu.VMEM((BLK,), jnp.int32),
            pltpu.VMEM((BLK,), jnp.int32),
            pltpu.VMEM((BLK,), jnp.float32),
            pltpu.VMEM((BLK,), jnp.float32),
            pltpu.VMEM((BLK,), jnp.float32),
            pltpu.VMEM((BLK,), jnp.float32),
            pltpu.SemaphoreType.DMA,
            pltpu.SemaphoreType.DMA,
            pltpu.SemaphoreType.DMA,
            pltpu.SemaphoreType.DMA,
            pltpu.VMEM_SHARED((N_PAD,), jnp.float32),
            pltpu.VMEM_SHARED((N_PAD,), jnp.float32),
        ],
    )
    return f(x1, y1, eflat, aflat)


def _loss_body(o0_ref, o1_ref, a0_ref, a1_ref, r0_ref, r1_ref, out_ref):
    a = a0_ref[...] + a1_ref[...]
    r = r0_ref[...] + r1_ref[...]
    out_ref[0, 0] = jnp.sum(jnp.abs(o0_ref[...] - a) + jnp.abs(o1_ref[...] - r))


def _final_loss(o0, o1, a0, a1, r0, r1):
    return pl.pallas_call(
        _loss_body,
        out_shape=jax.ShapeDtypeStruct((1, 1), jnp.float32),
        out_specs=pl.BlockSpec(memory_space=pltpu.SMEM),
    )(o0, o1, a0, a1, r0, r1)


@jax.jit
def kernel(inputs, output, edges, attributes):
    del inputs
    n = output.shape[0]
    pad = N_PAD - n
    v2 = jnp.pad(output[:, 2], (0, pad)).reshape(ROWS, 128)
    th2 = jnp.pad(output[:, 3], (0, pad)).reshape(ROWS, 128)
    x2, y2 = _node_xy(v2, th2)

    e32 = edges.astype(jnp.int32)
    eflat = e32.reshape(2, CHUNKS, 128).transpose(1, 0, 2).reshape(-1)
    aflat = attributes.reshape(CHUNKS, 128, 4).transpose(0, 2, 1).reshape(-1)

    pa0, pa1, pr0, pr1 = _edge_accumulate(
        x2.reshape(-1), y2.reshape(-1), eflat, aflat)

    o0 = jnp.pad(output[:, 0], (0, pad)).reshape(ROWS, 128)
    o1 = jnp.pad(output[:, 1], (0, pad)).reshape(ROWS, 128)
    rs = (ROWS, 128)
    loss = _final_loss(o0, o1, pa0.reshape(rs), pa1.reshape(rs),
                       pr0.reshape(rs), pr1.reshape(rs))
    return loss[0, 0]
